# baseline (device time: 17164 ns/iter reference)
import jax
import jax.numpy as jnp
from jax import lax
from jax.experimental import pallas as pl
from jax.experimental.pallas import tpu as pltpu

N_Z = 4
N_XY = 8
N_SPLIT = 8
N_CHUNKS = 4


def kernel(x, dy, gamma):
    m, d = x.shape
    rows = m // N_SPLIT

    def body(x_hbm, dy_hbm, gamma_hbm, out_ref, xv, dyv, own_z, own_xy,
             zsl, xysl, copy_sems, z_send, z_recv, xy_send, xy_recv):
        my_x = lax.axis_index("x")
        my_y = lax.axis_index("y")
        my_z = lax.axis_index("z")
        xy = my_x * 4 + my_y
        row0 = xy * rows

        def xy_peer(dxy):
            t = lax.rem(xy + dxy, N_XY)
            return t // 4, lax.rem(t, 4)

        def z_rdma(c, off):
            return pltpu.make_async_remote_copy(
                src_ref=own_z.at[c],
                dst_ref=zsl.at[c, off - 1],
                send_sem=z_send.at[3 * c + off - 1],
                recv_sem=z_recv.at[3 * c + off - 1],
                device_id=(my_x, my_y, lax.rem(my_z + off, N_Z)),
                device_id_type=pl.DeviceIdType.MESH,
            )

        def xy_rdma(c, dxy):
            tx, ty = xy_peer(dxy)
            return pltpu.make_async_remote_copy(
                src_ref=own_xy.at[c],
                dst_ref=xysl.at[c, dxy - 1],
                send_sem=xy_send.at[7 * c + dxy - 1],
                recv_sem=xy_recv.at[7 * c + dxy - 1],
                device_id=(tx, ty, my_z),
                device_id_type=pl.DeviceIdType.MESH,
            )

        chunk = rows // N_CHUNKS
        copies = []
        for c in range(N_CHUNKS):
            r0 = row0 + c * chunk
            cx = pltpu.make_async_copy(
                x_hbm.at[pl.ds(r0, chunk), :],
                xv.at[pl.ds(c * chunk, chunk), :], copy_sems.at[2 * c])
            cy = pltpu.make_async_copy(
                dy_hbm.at[pl.ds(r0, chunk), :],
                dyv.at[pl.ds(c * chunk, chunk), :], copy_sems.at[2 * c + 1])
            cx.start()
            cy.start()
            copies.append((cx, cy))

        barrier_sem = pltpu.get_barrier_semaphore()
        for off in (1, 2, 3):
            pl.semaphore_signal(
                barrier_sem, inc=1,
                device_id=(my_x, my_y, lax.rem(my_z + off, N_Z)),
                device_id_type=pl.DeviceIdType.MESH,
            )
        for dxy in range(1, N_XY):
            tx, ty = xy_peer(dxy)
            pl.semaphore_signal(
                barrier_sem, inc=1,
                device_id=(tx, ty, my_z),
                device_id_type=pl.DeviceIdType.MESH,
            )

        partials = []
        for c in range(N_CHUNKS):
            cx, cy = copies[c]
            cx.wait()
            cy.wait()
            sl = pl.ds(c * chunk, chunk)
            xb = xv[sl, :]
            dyb = dyv[sl, :]
            s1 = jnp.sum(xb, axis=1)
            s2 = jnp.sum(xb * xb, axis=1)
            mu = s1 / d
            var = s2 / d - mu * mu
            rstd = lax.rsqrt(var + 1e-5)
            t = xb * dyb
            w1 = rstd.reshape(1, chunk)
            w2 = jnp.stack([mu * rstd, jnp.ones_like(mu)])
            a = jnp.dot(w1, t, preferred_element_type=jnp.float32)
            b = jnp.dot(w2, dyb, preferred_element_type=jnp.float32)
            partial = jnp.concatenate([a - b[0:1], b[1:2]], axis=0)
            partials.append(partial)
            own_z[c] = partial.astype(jnp.bfloat16)
            if c == 0:
                pl.semaphore_wait(barrier_sem, 10)
            for off in (1, 2, 3):
                z_rdma(c, off).start()

        colsums = []
        for c in range(N_CHUNKS):
            for off in (1, 2, 3):
                z_rdma(c, off).wait_recv()
            zs = zsl[c].astype(jnp.float32)
            colsum = partials[c] + zs[0] + zs[1] + zs[2]
            colsums.append(colsum)
            own_xy[c] = colsum.astype(jnp.bfloat16)
            for dxy in range(1, N_XY):
                xy_rdma(c, dxy).start()

        total = colsums[0] + colsums[1] + colsums[2] + colsums[3]
        for c in range(N_CHUNKS):
            for dxy in range(1, N_XY):
                xy_rdma(c, dxy).wait_recv()
            total = total + jnp.sum(xysl[c].astype(jnp.float32), axis=0)
        out_ref[...] = total

        for c in range(N_CHUNKS):
            for off in (1, 2, 3):
                z_rdma(c, off).wait_send()
            for dxy in range(1, N_XY):
                xy_rdma(c, dxy).wait_send()

    return pl.pallas_call(
        body,
        in_specs=[
            pl.BlockSpec(memory_space=pltpu.MemorySpace.HBM),
            pl.BlockSpec(memory_space=pltpu.MemorySpace.HBM),
            pl.BlockSpec(memory_space=pltpu.MemorySpace.HBM),
        ],
        out_specs=pl.BlockSpec(memory_space=pltpu.MemorySpace.VMEM),
        out_shape=jax.ShapeDtypeStruct((2, d), jnp.float32),
        scratch_shapes=[
            pltpu.VMEM((rows, d), jnp.float32),
            pltpu.VMEM((rows, d), jnp.float32),
            pltpu.VMEM((N_CHUNKS, 2, d), jnp.bfloat16),
            pltpu.VMEM((N_CHUNKS, 2, d), jnp.bfloat16),
            pltpu.VMEM((N_CHUNKS, 3, 2, d), jnp.bfloat16),
            pltpu.VMEM((N_CHUNKS, 7, 2, d), jnp.bfloat16),
            pltpu.SemaphoreType.DMA((2 * N_CHUNKS,)),
            pltpu.SemaphoreType.DMA((3 * N_CHUNKS,)),
            pltpu.SemaphoreType.DMA((3 * N_CHUNKS,)),
            pltpu.SemaphoreType.DMA((7 * N_CHUNKS,)),
            pltpu.SemaphoreType.DMA((7 * N_CHUNKS,)),
        ],
        compiler_params=pltpu.CompilerParams(
            collective_id=0,
        ),
    )(x, dy, gamma)
